# baseline (speedup 1.0000x reference)
import jax
import jax.numpy as jnp
from jax import lax
from jax.experimental import pallas as pl
from jax.experimental.pallas import tpu as pltpu

_NE = 8192
_D = 256
_N = 16384
_DECAY = 0.9
_EPS = 1e-5

_ST = 512
_T = 256
_KB = 512
_SUB = 64
_AD = 256


def _stats_body(x_ref, stats_ref, acc_ref):
    phase = pl.program_id(0)
    step = pl.program_id(1)
    nsteps = pl.num_programs(1)
    x = x_ref[...]

    @pl.when(jnp.logical_and(phase == 0, step == 0))
    def _():
        acc_ref[...] = jnp.zeros_like(acc_ref)

    @pl.when(phase == 0)
    def _():
        acc_ref[0:1, :] += jnp.sum(x, axis=0, keepdims=True)

    @pl.when(phase == 1)
    def _():
        mean = acc_ref[0:1, :] * (1.0 / _N)
        d = x - mean
        acc_ref[1:2, :] += jnp.sum(d * d, axis=0, keepdims=True)

    @pl.when(jnp.logical_and(phase == 1, step == nsteps - 1))
    def _():
        stats_ref[...] = acc_ref[...] * (1.0 / _N)


def _argmin_body(stats_ref, x_ref, e_ref, xa_ref, idx_ref, cnt_ref, e2_ref):
    s = pl.program_id(0)

    @pl.when(s == 0)
    def _():
        for b in range(_NE // 256):
            eb = e_ref[pl.ds(b * 256, 256), :]
            e2_ref[pl.ds(b * 256, 256), :] = jnp.sum(
                eb * eb, axis=1, keepdims=True)

    mean = stats_ref[0:1, :]
    var = stats_ref[1:2, :]
    rstd = lax.rsqrt(var + _EPS)
    xn = (x_ref[...] - mean) * rstd
    x2 = jnp.sum(xn * xn, axis=1)[None, :]
    xa_ref[...] = xn

    def jstep(j, carry):
        rmin, ridx = carry
        eb = e_ref[pl.ds(j * _KB, _KB), :]
        dot = lax.dot_general(eb, xn, (((1,), (1,)), ((), ())),
                              preferred_element_type=jnp.float32)
        e2 = e2_ref[pl.ds(j * _KB, _KB), :]
        base = j * _KB
        for c in range(_KB // _SUB):
            d = (x2 + e2[c * _SUB:(c + 1) * _SUB, :]
                 - 2.0 * dot[c * _SUB:(c + 1) * _SUB, :])
            cmin = jnp.min(d, axis=0, keepdims=True)
            ciota = (lax.broadcasted_iota(jnp.int32, (_SUB, _T), 0)
                     + (base + c * _SUB))
            cidx = jnp.min(jnp.where(d == cmin, ciota, _NE), axis=0,
                           keepdims=True)
            upd = cmin < rmin
            rmin = jnp.where(upd, cmin, rmin)
            ridx = jnp.where(upd, cidx, ridx)
        return rmin, ridx

    rmin0 = jnp.full((1, _T), 3.0e38, jnp.float32)
    ridx0 = jnp.full((1, _T), _NE, jnp.int32)
    _, ridx = lax.fori_loop(0, _NE // _KB, jstep, (rmin0, ridx0))
    idx_ref[...] = ridx[None]

    @pl.when(s == 0)
    def _():
        for b in range(_NE // 256):
            cnt_ref[pl.ds(b * 256, 256), :] = jnp.zeros(
                (256, 1), jnp.float32)

    def cstep(j, carry):
        for c in range(_KB // _SUB):
            ciota = (lax.broadcasted_iota(jnp.int32, (_SUB, _T), 0)
                     + (j * _KB + c * _SUB))
            eq = (ciota == ridx).astype(jnp.float32)
            cnt_ref[pl.ds(j * _KB + c * _SUB, _SUB), :] += jnp.sum(
                eq, axis=1, keepdims=True)
        return carry

    lax.fori_loop(0, _NE // _KB, cstep, 0)


def _dw_body(idx_ref, xa_ref, dw_ref, oh_ref):
    s = pl.program_id(0)
    ridx = idx_ref[0]

    @pl.when(s == 0)
    def _():
        for b in range(_NE // 256):
            dw_ref[pl.ds(b * 256, 256), :] = jnp.zeros(
                (256, _D), jnp.float32)

    xnb = xa_ref[...].astype(jnp.bfloat16)

    def jstep(j, carry):
        for c in range(_KB // _SUB):
            ciota = (lax.broadcasted_iota(jnp.int32, (_SUB, _T), 0)
                     + (j * _KB + c * _SUB))
            oh_ref[pl.ds(c * _SUB, _SUB), :] = (
                ciota == ridx).astype(jnp.bfloat16)
        dwp = lax.dot_general(oh_ref[...], xnb,
                              (((1,), (0,)), ((), ())),
                              preferred_element_type=jnp.float32)
        dw_ref[pl.ds(j * _KB, _KB), :] += dwp
        return carry

    lax.fori_loop(0, _NE // _KB, jstep, 0)


def _final_body(stats_ref, e_ref, clus_ref, dwa_ref, cnt_ref, out_ref):
    mean = stats_ref[0:1, :]
    var = stats_ref[1:2, :]
    counts = cnt_ref[...]
    dw = dwa_ref[:, 0:_D]
    clus = clus_ref[...]
    cs = clus * _DECAY + (1.0 - _DECAY) * counts
    emb = (clus * e_ref[...] * _DECAY + (1.0 - _DECAY) * dw) / cs
    run_std = jnp.sqrt(var * (_N / (_N - 1.0)) + _EPS)
    out_ref[...] = emb * run_std + mean


@jax.jit
def _run(x, vq_embedding, vq_cluster_size):
    stats = pl.pallas_call(
        _stats_body,
        grid=(2, _N // _ST),
        in_specs=[pl.BlockSpec((_ST, _D), lambda p, s: (s, 0))],
        out_specs=pl.BlockSpec((2, _D), lambda p, s: (0, 0)),
        out_shape=jax.ShapeDtypeStruct((2, _D), jnp.float32),
        scratch_shapes=[pltpu.VMEM((2, _D), jnp.float32)],
    )(x)

    xa, idx3, cnt = pl.pallas_call(
        _argmin_body,
        grid=(_N // _T,),
        in_specs=[
            pl.BlockSpec((2, _D), lambda s: (0, 0)),
            pl.BlockSpec((_T, _D), lambda s: (s, 0)),
            pl.BlockSpec((_NE, _D), lambda s: (0, 0)),
        ],
        out_specs=[
            pl.BlockSpec((_T, _AD), lambda s: (s, 0)),
            pl.BlockSpec((1, 1, _T), lambda s: (s, 0, 0)),
            pl.BlockSpec((_NE, 1), lambda s: (0, 0)),
        ],
        out_shape=[
            jax.ShapeDtypeStruct((_N, _AD), jnp.float32),
            jax.ShapeDtypeStruct((_N // _T, 1, _T), jnp.int32),
            jax.ShapeDtypeStruct((_NE, 1), jnp.float32),
        ],
        scratch_shapes=[
            pltpu.VMEM((_NE, 1), jnp.float32),
        ],
        compiler_params=pltpu.CompilerParams(
            vmem_limit_bytes=100 * 1024 * 1024,
        ),
    )(stats, x, vq_embedding)

    dwa = pl.pallas_call(
        _dw_body,
        grid=(_N // _T,),
        in_specs=[
            pl.BlockSpec((1, 1, _T), lambda s: (s, 0, 0)),
            pl.BlockSpec((_T, _AD), lambda s: (s, 0)),
        ],
        out_specs=pl.BlockSpec((_NE, _D), lambda s: (0, 0)),
        out_shape=jax.ShapeDtypeStruct((_NE, _D), jnp.float32),
        scratch_shapes=[pltpu.VMEM((_KB, _T), jnp.bfloat16)],
        compiler_params=pltpu.CompilerParams(
            vmem_limit_bytes=100 * 1024 * 1024,
        ),
    )(idx3, xa)

    clus2d = vq_cluster_size.reshape(_NE, 1)
    _B = 512
    out = pl.pallas_call(
        _final_body,
        grid=(_NE // _B,),
        in_specs=[
            pl.BlockSpec((2, _D), lambda b: (0, 0)),
            pl.BlockSpec((_B, _D), lambda b: (b, 0)),
            pl.BlockSpec((_B, 1), lambda b: (b, 0)),
            pl.BlockSpec((_B, _AD), lambda b: (b, 0)),
            pl.BlockSpec((_B, 1), lambda b: (b, 0)),
        ],
        out_specs=pl.BlockSpec((_B, _D), lambda b: (b, 0)),
        out_shape=jax.ShapeDtypeStruct((_NE, _D), jnp.float32),
    )(stats, vq_embedding, clus2d, dwa, cnt)
    return out


def kernel(x, vq_embedding, vq_cluster_size):
    return _run(x, vq_embedding, vq_cluster_size)
